# baseline (speedup 1.0000x reference)
import functools

import jax
import jax.numpy as jnp
from jax import lax
from jax.experimental import pallas as pl
from jax.experimental.pallas import tpu as pltpu
from jax.experimental.pallas import tpu_sc as plsc

_B, _N, _C_IN, _C_OUT = 4, 1024, 768, 384
_G = _N // 8
_EW = _N // 16


def _knn_body(x_ref, theta_ref, xt_ref, g1_ref, g2_ref, g3_ref):
    b = pl.program_id(0)
    xb = x_ref[0]

    inner = -2.0 * jnp.dot(xb, xb.T)
    sq = jnp.sum(xb * xb, axis=1, keepdims=True)
    dis = sq + inner + sq.T

    col = jax.lax.broadcasted_iota(jnp.int32, (_N, _N), 1)

    m1 = jnp.min(dis, axis=1, keepdims=True)
    i1 = jnp.min(jnp.where(dis == m1, col, _N), axis=1, keepdims=True)
    dis2 = jnp.where(col == i1, jnp.inf, dis)
    m2 = jnp.min(dis2, axis=1, keepdims=True)
    i2 = jnp.min(jnp.where(dis2 == m2, col, _N), axis=1, keepdims=True)

    xt_ref[0] = jnp.dot(xb, theta_ref[...])

    rowvec = jax.lax.broadcasted_iota(jnp.int32, (_N, 1), 0)
    sel = (i1 != rowvec) & (i2 != rowvec)
    g1_ref[0] = (i1 + b * _N).T
    g2_ref[0] = (i2 + b * _N).T
    g3_ref[0] = jnp.where(sel, rowvec + b * _N, _B * _N).T


def _make_sc_gather():
    mesh = plsc.VectorSubcoreMesh(core_axis_name="c", subcore_axis_name="s")

    @functools.partial(
        pl.kernel,
        mesh=mesh,
        out_type=jax.ShapeDtypeStruct((_B * _N, _C_OUT), jnp.float32),
        scratch_types=[
            pltpu.VMEM((_EW,), jnp.int32),
            pltpu.VMEM((_EW,), jnp.int32),
            pltpu.VMEM((_EW,), jnp.int32),
            pltpu.VMEM((_EW, _C_OUT), jnp.float32),
            pltpu.VMEM((_EW, _C_OUT), jnp.float32),
        ],
    )
    def sc_gather(xt_hbm, g1_hbm, g2_hbm, g3_hbm, xe_out, l1, l2, l3, r1, r2):
        cid = lax.axis_index("c")
        sid = lax.axis_index("s")
        base = sid * _EW
        nch = _C_OUT // 16

        def accumulate(j, carry):
            for t in range(nch):
                r1[j, pl.ds(t * 16, 16)] = (r1[j, pl.ds(t * 16, 16)]
                                            + r2[j, pl.ds(t * 16, 16)])
            return carry

        for bi in range(2):
            b = cid * 2 + bi
            gbase = b * _N + base
            pltpu.sync_copy(g1_hbm.at[pl.ds(gbase, _EW)], l1)
            pltpu.sync_copy(g2_hbm.at[pl.ds(gbase, _EW)], l2)
            pltpu.sync_copy(g3_hbm.at[pl.ds(gbase, _EW)], l3)

            pltpu.sync_copy(xt_hbm.at[l1], r1)
            pltpu.sync_copy(xt_hbm.at[l2], r2)
            lax.fori_loop(0, _EW, accumulate, 0)
            pltpu.sync_copy(xt_hbm.at[l3], r2)
            lax.fori_loop(0, _EW, accumulate, 0)

            pltpu.sync_copy(r1, xe_out.at[pl.ds(gbase, _EW)])

    return sc_gather


_sc_gather_cache = []


def _sc_gather(*args):
    if not _sc_gather_cache:
        _sc_gather_cache.append(_make_sc_gather())
    return _sc_gather_cache[0](*args)


def _stats(xn):
    g = xn.reshape(_G, 8, _C_OUT)
    rsum = jnp.sum(g, axis=2)
    p2a = jnp.sum(g[:, 2, 0:256], axis=1, keepdims=True)
    p5a = jnp.sum(g[:, 5, 0:128], axis=1, keepdims=True)
    s0 = rsum[:, 0:1] + rsum[:, 1:2] + p2a
    s1 = (rsum[:, 2:3] - p2a) + rsum[:, 3:4] + rsum[:, 4:5] + p5a
    s2 = (rsum[:, 5:6] - p5a) + rsum[:, 6:7] + rsum[:, 7:8]
    return s0, s1, s2


def _agg_bn_body(xe_ref, g1_ref, g2_ref, bias_ref, w_ref, b_ref,
                 out_ref, acc_ref):
    b = pl.program_id(0)

    @pl.when(b == 0)
    def _zero():
        acc_ref[...] = jnp.zeros_like(acc_ref)

    xesum = xe_ref[0]
    i1 = g1_ref[0].T - b * _N
    i2 = g2_ref[0].T - b * _N
    bias = bias_ref[...]

    rowvec = jax.lax.broadcasted_iota(jnp.int32, (_N, 1), 0)
    de = (3.0
          - (i1 == rowvec).astype(jnp.float32)
          - (i2 == rowvec).astype(jnp.float32))
    xe = xesum / de

    col = jax.lax.broadcasted_iota(jnp.int32, (_N, _N), 1)
    row = jax.lax.broadcasted_iota(jnp.int32, (_N, _N), 0)
    h = ((row == i1.T) | (row == i2.T) | (row == col)).astype(jnp.float32)
    dn = jnp.sum(h, axis=1, keepdims=True)
    xn = jnp.dot(h, xe) / dn + bias
    out_ref[b] = xn

    s0, s1, s2 = _stats(xn)
    q0, q1, q2 = _stats(xn * xn)
    acc_ref[:, 0:1] += s0
    acc_ref[:, 1:2] += s1
    acc_ref[:, 2:3] += s2
    acc_ref[:, 4:5] += q0
    acc_ref[:, 5:6] += q1
    acc_ref[:, 6:7] += q2

    @pl.when(b == _B - 1)
    def _normalize():
        cnt = jnp.float32(_B * _N)
        w2 = w_ref[...]
        b2 = b_ref[...]
        r3 = jax.lax.broadcasted_iota(jnp.int32, (_G, 8, _C_OUT), 1)
        c3 = jax.lax.broadcasted_iota(jnp.int32, (_G, 8, _C_OUT), 2)
        ch = (r3 * _C_OUT + c3) // _N

        def full(t):
            mean = acc_ref[:, t:t + 1] / cnt
            var = acc_ref[:, t + 4:t + 5] / cnt - mean * mean
            scale = w2[:, t:t + 1] / jnp.sqrt(var + 1e-5)
            shift = b2[:, t:t + 1] - mean * scale
            return scale[:, :, None], shift[:, :, None]

        sc0, sh0 = full(0)
        sc1, sh1 = full(1)
        sc2, sh2 = full(2)
        scalef = jnp.where(ch == 0, sc0, jnp.where(ch == 1, sc1, sc2))
        shiftf = jnp.where(ch == 0, sh0, jnp.where(ch == 1, sh1, sh2))
        for bb in range(_B):
            v = out_ref[bb].reshape(_G, 8, _C_OUT)
            y = jnp.maximum(v * scalef + shiftf, 0.0)
            out_ref[bb] = y.reshape(_N, _C_OUT)


def kernel(x, theta, bias, bn_weight, bn_bias):
    xt, g1, g2, g3 = pl.pallas_call(
        _knn_body,
        grid=(_B,),
        in_specs=[
            pl.BlockSpec((1, _N, _C_IN), lambda b: (b, 0, 0)),
            pl.BlockSpec((_C_IN, _C_OUT), lambda b: (0, 0)),
        ],
        out_specs=[
            pl.BlockSpec((1, _N, _C_OUT), lambda b: (b, 0, 0)),
            pl.BlockSpec((1, 1, _N), lambda b: (b, 0, 0)),
            pl.BlockSpec((1, 1, _N), lambda b: (b, 0, 0)),
            pl.BlockSpec((1, 1, _N), lambda b: (b, 0, 0)),
        ],
        out_shape=[
            jax.ShapeDtypeStruct((_B, _N, _C_OUT), jnp.float32),
            jax.ShapeDtypeStruct((_B, 1, _N), jnp.int32),
            jax.ShapeDtypeStruct((_B, 1, _N), jnp.int32),
            jax.ShapeDtypeStruct((_B, 1, _N), jnp.int32),
        ],
    )(x, theta)

    xt_pad = jnp.concatenate(
        [xt.reshape(_B * _N, _C_OUT),
         jnp.zeros((8, _C_OUT), jnp.float32)], axis=0)
    xe = _sc_gather(xt_pad, g1.reshape(_B * _N), g2.reshape(_B * _N),
                    g3.reshape(_B * _N))

    out = pl.pallas_call(
        _agg_bn_body,
        grid=(_B,),
        in_specs=[
            pl.BlockSpec((1, _N, _C_OUT), lambda b: (b, 0, 0)),
            pl.BlockSpec((1, 1, _N), lambda b: (b, 0, 0)),
            pl.BlockSpec((1, 1, _N), lambda b: (b, 0, 0)),
            pl.BlockSpec((1, _C_OUT), lambda b: (0, 0)),
            pl.BlockSpec((_G, 3), lambda b: (0, 0)),
            pl.BlockSpec((_G, 3), lambda b: (0, 0)),
        ],
        out_specs=pl.BlockSpec((_B, _N, _C_OUT), lambda b: (0, 0, 0)),
        out_shape=jax.ShapeDtypeStruct((_B, _N, _C_OUT), jnp.float32),
        scratch_shapes=[pltpu.VMEM((_G, 8), jnp.float32)],
    )(xe.reshape(_B, _N, _C_OUT), g1, g2,
      bias.reshape(1, _C_OUT), bn_weight.reshape(_G, 3), bn_bias.reshape(_G, 3))

    return out


# speedup vs baseline: 5.6871x; 5.6871x over previous
import jax
import jax.numpy as jnp
from jax.experimental import pallas as pl
from jax.experimental.pallas import tpu as pltpu

_B, _N, _C_IN, _C_OUT = 4, 1024, 768, 384
_G = _N // 8


def _stats(xn):
    g = xn.reshape(_G, 8, _C_OUT)
    rsum = jnp.sum(g, axis=2)
    p2a = jnp.sum(g[:, 2, 0:256], axis=1, keepdims=True)
    p5a = jnp.sum(g[:, 5, 0:128], axis=1, keepdims=True)
    s0 = rsum[:, 0:1] + rsum[:, 1:2] + p2a
    s1 = (rsum[:, 2:3] - p2a) + rsum[:, 3:4] + rsum[:, 4:5] + p5a
    s2 = (rsum[:, 5:6] - p5a) + rsum[:, 6:7] + rsum[:, 7:8]
    return s0, s1, s2


def _body(x_ref, theta_ref, bias_ref, w_ref, b_ref, out_ref, acc_ref):
    b = pl.program_id(0)

    @pl.when(b == 0)
    def _zero():
        acc_ref[...] = jnp.zeros_like(acc_ref)

    xb = x_ref[0]
    theta = theta_ref[...]
    bias = bias_ref[...]

    inner = -2.0 * jnp.dot(xb, xb.T)
    sq = jnp.sum(xb * xb, axis=1, keepdims=True)
    dis = sq + inner + sq.T

    col = jax.lax.broadcasted_iota(jnp.int32, (_N, _N), 1)
    row = jax.lax.broadcasted_iota(jnp.int32, (_N, _N), 0)

    m1 = jnp.min(dis, axis=1, keepdims=True)
    i1 = jnp.min(jnp.where(dis == m1, col, _N), axis=1, keepdims=True)
    dis2 = jnp.where(col == i1, jnp.inf, dis)
    m2 = jnp.min(dis2, axis=1, keepdims=True)
    i2 = jnp.min(jnp.where(dis2 == m2, col, _N), axis=1, keepdims=True)

    h = ((row == i1.T) | (row == i2.T) | (row == col)).astype(jnp.float32)
    ht = ((col == i1) | (col == i2) | (col == row)).astype(jnp.float32)

    rowvec = jax.lax.broadcasted_iota(jnp.int32, (_N, 1), 0)
    de = (3.0
          - (i1 == rowvec).astype(jnp.float32)
          - (i2 == rowvec).astype(jnp.float32))

    xt = jnp.dot(xb, theta)
    xe = jnp.dot(ht, xt) / de
    dn = jnp.sum(h, axis=1, keepdims=True)
    xn = jnp.dot(h, xe) / dn + bias

    out_ref[b] = xn

    s0, s1, s2 = _stats(xn)
    q0, q1, q2 = _stats(xn * xn)
    acc_ref[:, 0:1] += s0
    acc_ref[:, 1:2] += s1
    acc_ref[:, 2:3] += s2
    acc_ref[:, 4:5] += q0
    acc_ref[:, 5:6] += q1
    acc_ref[:, 6:7] += q2

    @pl.when(b == _B - 1)
    def _normalize():
        cnt = jnp.float32(_B * _N)
        w2 = w_ref[...]
        b2 = b_ref[...]
        r3 = jax.lax.broadcasted_iota(jnp.int32, (_G, 8, _C_OUT), 1)
        c3 = jax.lax.broadcasted_iota(jnp.int32, (_G, 8, _C_OUT), 2)
        ch = (r3 * _C_OUT + c3) // _N

        def full(t):
            mean = acc_ref[:, t:t + 1] / cnt
            var = acc_ref[:, t + 4:t + 5] / cnt - mean * mean
            scale = w2[:, t:t + 1] / jnp.sqrt(var + 1e-5)
            shift = b2[:, t:t + 1] - mean * scale
            return scale[:, :, None], shift[:, :, None]

        sc0, sh0 = full(0)
        sc1, sh1 = full(1)
        sc2, sh2 = full(2)
        scalef = jnp.where(ch == 0, sc0, jnp.where(ch == 1, sc1, sc2))
        shiftf = jnp.where(ch == 0, sh0, jnp.where(ch == 1, sh1, sh2))
        for bb in range(_B):
            v = out_ref[bb].reshape(_G, 8, _C_OUT)
            y = jnp.maximum(v * scalef + shiftf, 0.0)
            out_ref[bb] = y.reshape(_N, _C_OUT)


def kernel(x, theta, bias, bn_weight, bn_bias):
    return pl.pallas_call(
        _body,
        grid=(_B,),
        in_specs=[
            pl.BlockSpec((1, _N, _C_IN), lambda b: (b, 0, 0)),
            pl.BlockSpec((_C_IN, _C_OUT), lambda b: (0, 0)),
            pl.BlockSpec((1, _C_OUT), lambda b: (0, 0)),
            pl.BlockSpec((_G, 3), lambda b: (0, 0)),
            pl.BlockSpec((_G, 3), lambda b: (0, 0)),
        ],
        out_specs=pl.BlockSpec((_B, _N, _C_OUT), lambda b: (0, 0, 0)),
        out_shape=jax.ShapeDtypeStruct((_B, _N, _C_OUT), jnp.float32),
        scratch_shapes=[pltpu.VMEM((_G, 8), jnp.float32)],
    )(x, theta, bias.reshape(1, _C_OUT),
      bn_weight.reshape(_G, 3), bn_bias.reshape(_G, 3))
